# baseline (speedup 1.0000x reference)
import jax
import jax.numpy as jnp
from jax import lax
from jax.experimental import pallas as pl
from jax.experimental.pallas import tpu as pltpu
from jax.experimental.pallas import tpu_sc as plsc

VOCAB = 100000
HIDDEN = 128
NC = 2
NS = 16
NW = NC * NS
ROW_BLOCK = 20000
N_ROW_BLOCKS = VOCAB // ROW_BLOCK

BATCH = 4096
SEQ = 50
PAD_S = 56
CHUNK_B = 2
CHUNK_IDS = CHUNK_B * PAD_S
SLOTS = 4
SUPER = SLOTS


def _transform_body(e_ref, w0_ref, w1_ref, t_ref):
    wc = lax.dot_general(
        w0_ref[...], w1_ref[...], (((0,), (1,)), ((), ())),
        preferred_element_type=jnp.float32)
    t_ref[...] = lax.dot_general(
        e_ref[...], wc, (((1,), (0,)), ((), ())),
        preferred_element_type=jnp.float32)


def _transform_table(embed, W0, W1):
    return pl.pallas_call(
        _transform_body,
        grid=(N_ROW_BLOCKS,),
        in_specs=[
            pl.BlockSpec((ROW_BLOCK, HIDDEN), lambda i: (i, 0)),
            pl.BlockSpec((HIDDEN, HIDDEN), lambda i: (0, 0)),
            pl.BlockSpec((HIDDEN, HIDDEN), lambda i: (0, 0)),
        ],
        out_specs=pl.BlockSpec((ROW_BLOCK, HIDDEN), lambda i: (i, 0)),
        out_shape=jax.ShapeDtypeStruct((VOCAB, HIDDEN), jnp.float32),
    )(embed, W0, W1)


def _make_gather_body(n_chunk):
    n_super = n_chunk // SUPER
    n_pair = n_super // 2
    b_per_tile = n_chunk * CHUNK_B

    def gather_body(tab_hbm, idx_hbm, out_hbm, idx_v, ring_a, ring_b,
                    sga, sgb, ssa, ssb):
        wid = lax.axis_index("s") * NC + lax.axis_index("c")
        pltpu.sync_copy(idx_hbm.at[wid], idx_v)
        base = wid * b_per_tile

        def fire_gathers(ring, sem, sup):
            for k in range(SLOTS):
                pltpu.async_copy(
                    tab_hbm.at[idx_v.at[sup * SUPER + k]], ring.at[k], sem)

        def wait_gathers(ring, sem, sup):
            for k in range(SLOTS):
                pltpu.make_async_copy(
                    tab_hbm.at[idx_v.at[sup * SUPER + k]], ring.at[k], sem
                ).wait()

        def store_args(ring, sup):
            for k in range(SLOTS):
                c = sup * SUPER + k
                yield ring.at[k, pl.ds(0, SEQ)], out_hbm.at[base + 2 * c]
                yield ring.at[k, pl.ds(PAD_S, SEQ)], out_hbm.at[base + 2 * c + 1]

        def fire_stores(ring, sem, sup):
            for src, dst in store_args(ring, sup):
                pltpu.async_copy(src, dst, sem)

        def wait_stores(ring, sem, sup):
            for src, dst in store_args(ring, sup):
                pltpu.make_async_copy(src, dst, sem).wait()

        fire_gathers(ring_a, sga, 0)
        fire_gathers(ring_b, sgb, 1)

        def body(i, carry):
            sup_a = 2 * i
            sup_b = 2 * i + 1
            wait_gathers(ring_a, sga, sup_a)
            fire_stores(ring_a, ssa, sup_a)
            wait_gathers(ring_b, sgb, sup_b)
            fire_stores(ring_b, ssb, sup_b)

            @pl.when(i < n_pair - 1)
            def _():
                wait_stores(ring_a, ssa, sup_a)
                fire_gathers(ring_a, sga, sup_a + 2)
                wait_stores(ring_b, ssb, sup_b)
                fire_gathers(ring_b, sgb, sup_b + 2)

            return carry

        lax.fori_loop(0, n_pair, body, 0)
        wait_stores(ring_a, ssa, n_super - 2)
        wait_stores(ring_b, ssb, n_super - 1)

    return gather_body


def _gather_rows(table, idx, n_batch):
    n_chunk = idx.shape[1]
    mesh = plsc.VectorSubcoreMesh(core_axis_name="c", subcore_axis_name="s")
    return pl.kernel(
        _make_gather_body(n_chunk),
        out_type=jax.ShapeDtypeStruct((n_batch, SEQ, HIDDEN), jnp.float32),
        mesh=mesh,
        scratch_types=[
            pltpu.VMEM((n_chunk, CHUNK_IDS), jnp.int32),
            pltpu.VMEM((SLOTS, CHUNK_IDS, HIDDEN), jnp.float32),
            pltpu.VMEM((SLOTS, CHUNK_IDS, HIDDEN), jnp.float32),
            pltpu.SemaphoreType.DMA,
            pltpu.SemaphoreType.DMA,
            pltpu.SemaphoreType.DMA,
            pltpu.SemaphoreType.DMA,
        ],
        compiler_params=pltpu.CompilerParams(use_tc_tiling_on_sc=True),
    )(table, idx)


def kernel(input_ids, embed, W0, W1):
    table = _transform_table(embed, W0, W1)
    ids = input_ids.astype(jnp.int32)
    pad = (jnp.arange(BATCH, dtype=jnp.int32)[:, None] * 6151
           + jnp.arange(PAD_S - SEQ, dtype=jnp.int32)[None, :] * 1031) % VOCAB
    ids = jnp.concatenate([ids, pad], axis=1)
    half = BATCH // 2
    n_chunk_half = half // NW // CHUNK_B
    outs = []
    for h in range(2):
        idx = ids[h * half:(h + 1) * half].reshape(NW, n_chunk_half, CHUNK_IDS)
        outs.append(_gather_rows(table, idx, half))
    return jnp.concatenate(outs, axis=0)


# speedup vs baseline: 1.4877x; 1.4877x over previous
import functools

import jax
import jax.numpy as jnp
from jax import lax
from jax.experimental import pallas as pl
from jax.experimental.pallas import tpu as pltpu
from jax.experimental.pallas import tpu_sc as plsc

VOCAB = 100000
HIDDEN = 128
NC = 2
NS = 16
NW = NC * NS
B_TOTAL = 4096 * 50
B_PER_W = B_TOTAL // NW
CHUNK = 128
N_CHUNKS = B_PER_W // CHUNK
ROW_BLOCK = 20000
N_ROW_BLOCKS = VOCAB // ROW_BLOCK


def _transform_body(e_ref, w0_ref, w1_ref, t_ref):
    wc = lax.dot_general(
        w0_ref[...], w1_ref[...], (((0,), (1,)), ((), ())),
        preferred_element_type=jnp.float32)
    t_ref[...] = lax.dot_general(
        e_ref[...], wc, (((1,), (0,)), ((), ())),
        preferred_element_type=jnp.float32)


def _transform_table(embed, W0, W1):
    return pl.pallas_call(
        _transform_body,
        grid=(N_ROW_BLOCKS,),
        in_specs=[
            pl.BlockSpec((ROW_BLOCK, HIDDEN), lambda i: (i, 0)),
            pl.BlockSpec((HIDDEN, HIDDEN), lambda i: (0, 0)),
            pl.BlockSpec((HIDDEN, HIDDEN), lambda i: (0, 0)),
        ],
        out_specs=pl.BlockSpec((ROW_BLOCK, HIDDEN), lambda i: (i, 0)),
        out_shape=jax.ShapeDtypeStruct((VOCAB, HIDDEN), jnp.float32),
    )(embed, W0, W1)


BATCH = 4096
SEQ = 50
PAD_S = 56
B_PER_TILE = BATCH // NW
IDX_PER_TILE = B_PER_TILE * PAD_S


CHUNK_B = 2
CHUNK_IDS = CHUNK_B * PAD_S
N_CHUNK = B_PER_TILE // CHUNK_B
SLOTS = 4
SUPER = SLOTS
N_SUPER = N_CHUNK // SUPER
N_PAIR = N_SUPER // 2


def _gather_body(tab_hbm, idx_hbm, out_hbm, idx_v, ring_a, ring_b, sga, sgb, ssa, ssb):
    wid = lax.axis_index("s") * NC + lax.axis_index("c")
    pltpu.sync_copy(idx_hbm.at[wid], idx_v)
    base = wid * B_PER_TILE

    def idx_sl(c):
        return idx_v.at[c]

    def fire_gathers(ring, sem, sup):
        for k in range(SLOTS):
            pltpu.async_copy(tab_hbm.at[idx_sl(sup * SUPER + k)], ring.at[k], sem)

    def wait_gathers(ring, sem, sup):
        for k in range(SLOTS):
            pltpu.make_async_copy(
                tab_hbm.at[idx_sl(sup * SUPER + k)], ring.at[k], sem).wait()

    def store_args(ring, sup):
        for k in range(SLOTS):
            c = sup * SUPER + k
            yield ring.at[k, pl.ds(0, SEQ)], out_hbm.at[base + 2 * c]
            yield ring.at[k, pl.ds(PAD_S, SEQ)], out_hbm.at[base + 2 * c + 1]

    def fire_stores(ring, sem, sup):
        for src, dst in store_args(ring, sup):
            pltpu.async_copy(src, dst, sem)

    def wait_stores(ring, sem, sup):
        for src, dst in store_args(ring, sup):
            pltpu.make_async_copy(src, dst, sem).wait()

    fire_gathers(ring_a, sga, 0)
    fire_gathers(ring_b, sgb, 1)

    def body(i, carry):
        sup_a = 2 * i
        sup_b = 2 * i + 1
        wait_gathers(ring_a, sga, sup_a)
        fire_stores(ring_a, ssa, sup_a)
        wait_gathers(ring_b, sgb, sup_b)
        fire_stores(ring_b, ssb, sup_b)

        @pl.when(i < N_PAIR - 1)
        def _():
            wait_stores(ring_a, ssa, sup_a)
            fire_gathers(ring_a, sga, sup_a + 2)
            wait_stores(ring_b, ssb, sup_b)
            fire_gathers(ring_b, sgb, sup_b + 2)

        return carry

    lax.fori_loop(0, N_PAIR, body, 0)
    wait_stores(ring_a, ssa, N_SUPER - 2)
    wait_stores(ring_b, ssb, N_SUPER - 1)


def _gather_rows(table, idx):
    mesh = plsc.VectorSubcoreMesh(core_axis_name="c", subcore_axis_name="s")
    return pl.kernel(
        _gather_body,
        out_type=jax.ShapeDtypeStruct((BATCH, SEQ, HIDDEN), jnp.float32),
        mesh=mesh,
        scratch_types=[
            pltpu.VMEM((N_CHUNK, CHUNK_IDS), jnp.int32),
            pltpu.VMEM((SLOTS, CHUNK_IDS, HIDDEN), jnp.float32),
            pltpu.VMEM((SLOTS, CHUNK_IDS, HIDDEN), jnp.float32),
            pltpu.SemaphoreType.DMA,
            pltpu.SemaphoreType.DMA,
            pltpu.SemaphoreType.DMA,
            pltpu.SemaphoreType.DMA,
        ],
        compiler_params=pltpu.CompilerParams(use_tc_tiling_on_sc=True),
    )(table, idx)


def kernel(input_ids, embed, W0, W1):
    table = _transform_table(embed, W0, W1)
    ids = input_ids.astype(jnp.int32)
    pad = (jnp.arange(BATCH, dtype=jnp.int32)[:, None] * 6151
           + jnp.arange(PAD_S - SEQ, dtype=jnp.int32)[None, :] * 1031) % VOCAB
    ids = jnp.concatenate([ids, pad], axis=1)
    idx = ids.reshape(NW, N_CHUNK, CHUNK_IDS)
    out = _gather_rows(table, idx)
    return out
